# Initial kernel scaffold; baseline (speedup 1.0000x reference)
import functools

import jax
import jax.numpy as jnp
from jax import lax
from jax.experimental import pallas as pl
from jax.experimental.pallas import tpu as pltpu
from jax.experimental.pallas import tpu_sc as plsc

NC = 2
NS = 16
CHUNK = 128
PAIRS_STEP = 2


def _sc_mesh():
    return plsc.VectorSubcoreMesh(core_axis_name="c", subcore_axis_name="s")


def _make_agg(n_nodes, d, n_chunks):
    n_pad = n_nodes + NS
    z_rows = n_pad // NS
    o_rows = n_nodes // NS
    per_core = n_chunks // NC
    per_sub = per_core // NS

    @functools.partial(
        pl.kernel,
        out_type=jax.ShapeDtypeStruct((NC, n_nodes, d), jnp.float32),
        mesh=_sc_mesh(),
        scratch_types=[
            pltpu.VMEM((2, CHUNK), jnp.int32),
            pltpu.VMEM((2, CHUNK), jnp.int32),
            pltpu.VMEM((CHUNK, d), jnp.float32),
            pltpu.VMEM((CHUNK, d), jnp.float32),
            pltpu.VMEM_SHARED((n_pad, d), jnp.float32),
            pltpu.SemaphoreType.DMA,
            pltpu.SemaphoreType.DMA,
        ],
    )
    def agg(g_hbm, idx_hbm, zeros_hbm, out_hbm,
            idx0, idx1, rows0, rows1, acc, sem0, sem1):
        c = lax.axis_index("c")
        s = lax.axis_index("s")
        pltpu.sync_copy(zeros_hbm, acc.at[pl.ds(s * z_rows, z_rows)])
        plsc.subcore_barrier()

        base = c * per_core + s * per_sub
        pltpu.sync_copy(idx_hbm.at[base], idx0)
        pltpu.async_copy(g_hbm.at[idx0.at[0]], rows0, sem0)

        @pl.loop(0, per_sub, step=PAIRS_STEP)
        def _(jj):
            pltpu.sync_copy(idx_hbm.at[base + jj + 1], idx1)
            d1 = pltpu.async_copy(g_hbm.at[idx1.at[0]], rows1, sem1)
            pltpu.make_async_copy(g_hbm.at[idx0.at[0]], rows0, sem0).wait()
            pltpu.sync_copy(rows0, acc.at[idx0.at[1]], add=True)

            @pl.when(jj + PAIRS_STEP < per_sub)
            def _():
                pltpu.sync_copy(idx_hbm.at[base + jj + 2], idx0)
                pltpu.async_copy(g_hbm.at[idx0.at[0]], rows0, sem0)

            d1.wait()
            pltpu.sync_copy(rows1, acc.at[idx1.at[1]], add=True)

        plsc.subcore_barrier()
        pltpu.sync_copy(acc.at[pl.ds(s * o_rows, o_rows)],
                        out_hbm.at[c, pl.ds(s * o_rows, o_rows)])

    return agg


def _make_hist(n_nodes, n_chunks):
    n_pad = n_nodes + NS
    z_rows = n_pad // NS
    o_rows = n_nodes // NS
    per_core = n_chunks // NC
    per_sub = per_core // NS
    w = 16

    @functools.partial(
        pl.kernel,
        out_type=jax.ShapeDtypeStruct((NC, n_nodes, w), jnp.float32),
        mesh=_sc_mesh(),
        scratch_types=[
            pltpu.VMEM((2, CHUNK), jnp.int32),
            pltpu.VMEM((CHUNK, w), jnp.float32),
            pltpu.VMEM_SHARED((n_pad, w), jnp.float32),
        ],
    )
    def hist(idx_hbm, ones_hbm, zeros_hbm, out_hbm, idx_v, ones_v, acc):
        c = lax.axis_index("c")
        s = lax.axis_index("s")
        pltpu.sync_copy(zeros_hbm, acc.at[pl.ds(s * z_rows, z_rows)])
        pltpu.sync_copy(ones_hbm, ones_v)
        plsc.subcore_barrier()

        base = c * per_core + s * per_sub

        @pl.loop(0, per_sub)
        def _(j):
            pltpu.sync_copy(idx_hbm.at[base + j], idx_v)
            pltpu.sync_copy(ones_v, acc.at[idx_v.at[1]], add=True)

        plsc.subcore_barrier()
        pltpu.sync_copy(acc.at[pl.ds(s * o_rows, o_rows)],
                        out_hbm.at[c, pl.ds(s * o_rows, o_rows)])

    return hist


def _mm1_body(x_ref, w_ref, o_ref):
    o_ref[...] = jax.lax.dot_general(
        x_ref[...], w_ref[...], (((1,), (0,)), ((), ())),
        precision=lax.Precision.HIGHEST, preferred_element_type=jnp.float32)


def _scale_body(h_ref, hist_ref, g_ref, dis_ref):
    deg = hist_ref[0, :, 0:1] + hist_ref[1, :, 0:1] + 1.0
    dis = lax.rsqrt(deg)
    dis_ref[...] = dis
    g_ref[...] = h_ref[...] * dis


def _post1_body(p_ref, g_ref, dis_ref, b_ref, w_ref, o_ref):
    dis = dis_ref[...]
    t = dis * (p_ref[0] + p_ref[1] + g_ref[...]) + b_ref[...]
    t = jnp.maximum(t, 0.0)
    h2 = jax.lax.dot_general(
        t, w_ref[...], (((1,), (0,)), ((), ())),
        precision=lax.Precision.HIGHEST, preferred_element_type=jnp.float32)
    o_ref[...] = h2 * dis


def _final_body(p_ref, g_ref, dis_ref, b_ref, o_ref):
    o = dis_ref[...] * (p_ref[0] + p_ref[1] + g_ref[...]) + b_ref[...]
    m = jnp.max(o, axis=1, keepdims=True)
    z = o - m
    lse = jnp.log(jnp.sum(jnp.exp(z), axis=1, keepdims=True))
    o_ref[...] = z - lse


def kernel(x, edge_index, W1, b1, W2, b2):
    n, d_in = x.shape
    d_hid = W1.shape[1]
    d_out = W2.shape[1]
    e = edge_index.shape[1]

    cpw = CHUNK * NC * NS * PAIRS_STEP
    e_pad = ((e + cpw - 1) // cpw) * cpw
    pad = e_pad - e
    n_chunks = e_pad // CHUNK
    row_p = jnp.concatenate(
        [edge_index[0], jnp.zeros((pad,), edge_index.dtype)])
    col_p = jnp.concatenate(
        [edge_index[1], jnp.full((pad,), n, edge_index.dtype)])
    idx = jnp.stack([row_p.reshape(n_chunks, CHUNK),
                     col_p.reshape(n_chunks, CHUNK)], axis=1).astype(jnp.int32)

    z_rows = (n + NS) // NS
    zeros_hid = jnp.zeros((z_rows, d_hid), jnp.float32)
    zeros_out = jnp.zeros((z_rows, d_out), jnp.float32)
    zeros_h16 = jnp.zeros((z_rows, 16), jnp.float32)
    ones_h16 = jnp.ones((CHUNK, 16), jnp.float32)

    hist = _make_hist(n, n_chunks)(idx, ones_h16, zeros_h16)

    h1 = pl.pallas_call(
        _mm1_body,
        out_shape=jax.ShapeDtypeStruct((n, d_hid), jnp.float32),
    )(x, W1)

    g1, dis = pl.pallas_call(
        _scale_body,
        out_shape=(jax.ShapeDtypeStruct((n, d_hid), jnp.float32),
                   jax.ShapeDtypeStruct((n, 1), jnp.float32)),
    )(h1, hist)

    p1 = _make_agg(n, d_hid, n_chunks)(g1, idx, zeros_hid)

    g2 = pl.pallas_call(
        _post1_body,
        out_shape=jax.ShapeDtypeStruct((n, d_out), jnp.float32),
    )(p1, g1, dis, b1.reshape(1, d_hid), W2)

    p2 = _make_agg(n, d_out, n_chunks)(g2, idx, zeros_out)

    out = pl.pallas_call(
        _final_body,
        out_shape=jax.ShapeDtypeStruct((n, d_out), jnp.float32),
    )(p2, g2, dis, b2.reshape(1, d_out))

    return out


# speedup vs baseline: 8.4591x; 8.4591x over previous
import dataclasses
import functools

import jax
import jax.numpy as jnp
from jax import lax
from jax.experimental import pallas as pl
from jax.experimental.pallas import tpu as pltpu
from jax.experimental.pallas import tpu_sc as plsc

NC = 2
NS = 16
CHUNK = 128
PAIRS_STEP = 2


def _sc_mesh():
    return plsc.VectorSubcoreMesh(core_axis_name="c", subcore_axis_name="s")


def _stripe(total):
    main = (total // NS) // 8 * 8
    tail = total - main * NS
    return main, tail


def _striped_copy(s, src_at, dst_at, total):
    main, tail = _stripe(total)
    pltpu.sync_copy(src_at(s * main, main), dst_at(s * main, main))
    if tail:
        @pl.when(s == NS - 1)
        def _():
            pltpu.sync_copy(src_at(NS * main, tail), dst_at(NS * main, tail))


def _make_agg(n_nodes, d, n_chunks):
    n_pad = n_nodes + NS
    per_core = n_chunks // NC
    per_sub = per_core // NS

    @functools.partial(
        pl.kernel,
        out_type=jax.ShapeDtypeStruct((NC, n_nodes, d), jnp.float32),
        mesh=_sc_mesh(),
        scratch_types=[
            pltpu.VMEM((2, CHUNK), jnp.int32),
            pltpu.VMEM((2, CHUNK), jnp.int32),
            pltpu.VMEM((CHUNK, d), jnp.float32),
            pltpu.VMEM((CHUNK, d), jnp.float32),
            pltpu.VMEM_SHARED((n_pad, d), jnp.float32),
            pltpu.SemaphoreType.DMA,
            pltpu.SemaphoreType.DMA,
        ],
    )
    def agg(g_hbm, idx_hbm, zeros_hbm, out_hbm,
            idx0, idx1, rows0, rows1, acc, sem0, sem1):
        c = lax.axis_index("c")
        s = lax.axis_index("s")
        _striped_copy(s, lambda o, l: zeros_hbm.at[pl.ds(0, l)],
                      lambda o, l: acc.at[pl.ds(o, l)], n_pad)
        plsc.subcore_barrier()

        base = c * per_core + s * per_sub
        pltpu.sync_copy(idx_hbm.at[base], idx0)
        pltpu.async_copy(g_hbm.at[idx0.at[0]], rows0, sem0)

        @pl.loop(0, per_sub, step=PAIRS_STEP)
        def _(jj):
            pltpu.sync_copy(idx_hbm.at[base + jj + 1], idx1)
            d1 = pltpu.async_copy(g_hbm.at[idx1.at[0]], rows1, sem1)
            pltpu.make_async_copy(g_hbm.at[idx0.at[0]], rows0, sem0).wait()
            pltpu.sync_copy(rows0, acc.at[idx0.at[1]], add=True)

            @pl.when(jj + PAIRS_STEP < per_sub)
            def _():
                pltpu.sync_copy(idx_hbm.at[base + jj + 2], idx0)
                pltpu.async_copy(g_hbm.at[idx0.at[0]], rows0, sem0)

            d1.wait()
            pltpu.sync_copy(rows1, acc.at[idx1.at[1]], add=True)

        plsc.subcore_barrier()
        _striped_copy(s, lambda o, l: acc.at[pl.ds(o, l)],
                      lambda o, l: out_hbm.at[c, pl.ds(o, l)], n_nodes)

    return agg


def _make_hist(n_nodes, n_chunks):
    n_pad = n_nodes + NS
    per_core = n_chunks // NC
    per_sub = per_core // NS

    cp = pltpu.CompilerParams()
    if "needs_layout_passes" in pltpu.CompilerParams.__dataclass_fields__:
        cp = dataclasses.replace(cp, needs_layout_passes=False)

    @functools.partial(
        pl.kernel,
        out_type=jax.ShapeDtypeStruct((NC * NS, n_pad), jnp.float32),
        mesh=_sc_mesh(),
        compiler_params=cp,
        scratch_types=[
            pltpu.VMEM((2, CHUNK), jnp.int32),
            pltpu.VMEM((n_pad,), jnp.float32),
        ],
    )
    def hist(idx_hbm, zeros_hbm, out_hbm, idx_v, lhist):
        c = lax.axis_index("c")
        s = lax.axis_index("s")
        pltpu.sync_copy(zeros_hbm, lhist)

        base = c * per_core + s * per_sub
        ones = jnp.full((16,), 1.0, jnp.float32)

        @pl.loop(0, per_sub)
        def _(j):
            pltpu.sync_copy(idx_hbm.at[base + j], idx_v)
            for l in range(CHUNK // 16):
                colv = idx_v[1, pl.ds(l * 16, 16)]
                plsc.addupdate_scatter(lhist, [colv], ones)

        pltpu.sync_copy(lhist, out_hbm.at[c * NS + s])

    return hist


def _mm1_body(x_ref, w_ref, o_ref):
    o_ref[...] = jax.lax.dot_general(
        x_ref[...], w_ref[...], (((1,), (0,)), ((), ())),
        precision=lax.Precision.HIGHEST, preferred_element_type=jnp.float32)


def _scale_body(h_ref, hist_ref, g_ref, dis_ref):
    deg = jnp.sum(hist_ref[...], axis=1, keepdims=True) + 1.0
    dis = lax.rsqrt(deg)
    dis_ref[...] = dis
    g_ref[...] = h_ref[...] * dis


def _post1_body(p_ref, g_ref, dis_ref, b_ref, w_ref, o_ref):
    dis = dis_ref[...]
    t = dis * (p_ref[0] + p_ref[1] + g_ref[...]) + b_ref[...]
    t = jnp.maximum(t, 0.0)
    h2 = jax.lax.dot_general(
        t, w_ref[...], (((1,), (0,)), ((), ())),
        precision=lax.Precision.HIGHEST, preferred_element_type=jnp.float32)
    d_out = h2.shape[1]
    o_ref[:, :d_out] = h2 * dis
    o_ref[:, d_out:] = jnp.zeros_like(h2)


def _final_body(p_ref, g_ref, dis_ref, b_ref, o_ref):
    d_out = o_ref.shape[1]
    o = dis_ref[...] * (p_ref[0, :, :d_out] + p_ref[1, :, :d_out]
                        + g_ref[:, :d_out]) + b_ref[...]
    m = jnp.max(o, axis=1, keepdims=True)
    z = o - m
    lse = jnp.log(jnp.sum(jnp.exp(z), axis=1, keepdims=True))
    o_ref[...] = z - lse


def kernel(x, edge_index, W1, b1, W2, b2):
    n, d_in = x.shape
    d_hid = W1.shape[1]
    d_out = W2.shape[1]
    e = edge_index.shape[1]

    cpw = CHUNK * NC * NS * PAIRS_STEP
    e_pad = ((e + cpw - 1) // cpw) * cpw
    pad = e_pad - e
    n_chunks = e_pad // CHUNK
    row_p = jnp.concatenate(
        [edge_index[0], jnp.zeros((pad,), edge_index.dtype)])
    col_p = jnp.concatenate(
        [edge_index[1], jnp.full((pad,), n, edge_index.dtype)])
    idx = jnp.stack([row_p.reshape(n_chunks, CHUNK),
                     col_p.reshape(n_chunks, CHUNK)], axis=1).astype(jnp.int32)

    z_rows, _ = _stripe(n + NS)
    zeros_hid = jnp.zeros((z_rows, d_hid), jnp.float32)
    zeros_flat = jnp.zeros((n + NS,), jnp.float32)

    hist32 = _make_hist(n, n_chunks)(idx, zeros_flat)
    hist_t = jnp.transpose(hist32)[:n]

    h1 = pl.pallas_call(
        _mm1_body,
        out_shape=jax.ShapeDtypeStruct((n, d_hid), jnp.float32),
    )(x, W1)

    g1, dis = pl.pallas_call(
        _scale_body,
        out_shape=(jax.ShapeDtypeStruct((n, d_hid), jnp.float32),
                   jax.ShapeDtypeStruct((n, 1), jnp.float32)),
    )(h1, hist_t)

    p1 = _make_agg(n, d_hid, n_chunks)(g1, idx, zeros_hid)

    g2 = pl.pallas_call(
        _post1_body,
        out_shape=jax.ShapeDtypeStruct((n, d_hid), jnp.float32),
    )(p1, g1, dis, b1.reshape(1, d_hid), W2)

    p2 = _make_agg(n, d_hid, n_chunks)(g2, idx, zeros_hid)

    out = pl.pallas_call(
        _final_body,
        out_shape=jax.ShapeDtypeStruct((n, d_out), jnp.float32),
    )(p2, g2, dis, b2.reshape(1, d_out))

    return out


# speedup vs baseline: 24.9612x; 2.9508x over previous
import dataclasses
import functools

import jax
import jax.numpy as jnp
from jax import lax
from jax.experimental import pallas as pl
from jax.experimental.pallas import tpu as pltpu
from jax.experimental.pallas import tpu_sc as plsc

NC = 2
NS = 16
CHUNK = 128
NBUF = 2
LOOKAHEAD = 1
NIDX = 2
PAIRS_STEP = NBUF * NIDX


def _sc_mesh():
    return plsc.VectorSubcoreMesh(core_axis_name="c", subcore_axis_name="s")


def _stripe(total):
    main = (total // NS) // 8 * 8
    tail = total - main * NS
    return main, tail


def _striped_copy(s, src_at, dst_at, total):
    main, tail = _stripe(total)
    pltpu.sync_copy(src_at(s * main, main), dst_at(s * main, main))
    if tail:
        @pl.when(s == NS - 1)
        def _():
            pltpu.sync_copy(src_at(NS * main, tail), dst_at(NS * main, tail))


def _make_agg(n_nodes, d, n_chunks):
    n_pad = n_nodes + NS
    per_core = n_chunks // NC
    per_sub = per_core // NS

    assert per_sub % (NBUF * NIDX) == 0
    per_half = per_sub // NIDX

    @functools.partial(
        pl.kernel,
        out_type=jax.ShapeDtypeStruct((NC, n_nodes, d), jnp.float32),
        mesh=_sc_mesh(),
        scratch_types=(
            [pltpu.VMEM((per_half, 2, CHUNK), jnp.int32)]
            + [pltpu.VMEM((CHUNK, d), jnp.float32)] * NBUF
            + [pltpu.VMEM_SHARED((n_pad, d), jnp.float32)]
            + [pltpu.SemaphoreType.DMA] * (2 * NBUF)
        ),
    )
    def agg(g_hbm, idx_hbm, zeros_hbm, out_hbm, idx_all, *scr):
        rows = scr[:NBUF]
        acc = scr[NBUF]
        gsems = scr[NBUF + 1:2 * NBUF + 1]
        ssems = scr[2 * NBUF + 1:]
        c = lax.axis_index("c")
        s = lax.axis_index("s")
        base = c * per_core + s * per_sub

        _striped_copy(s, lambda o, l: zeros_hbm.at[pl.ds(0, l)],
                      lambda o, l: acc.at[pl.ds(o, l)], n_pad)

        def gather(j, b):
            pltpu.async_copy(g_hbm.at[idx_all.at[j, 0]], rows[b], gsems[b])

        def gwait(j, b):
            pltpu.make_async_copy(
                g_hbm.at[idx_all.at[j, 0]], rows[b], gsems[b]).wait()

        def scat(j, b):
            pltpu.async_copy(rows[b], acc.at[idx_all.at[j, 1]], ssems[b],
                             add=True)

        def swait(j, b):
            pltpu.make_async_copy(
                rows[b], acc.at[idx_all.at[j, 1]], ssems[b]).wait()

        for h in range(NIDX):
            pltpu.sync_copy(
                idx_hbm.at[pl.ds(base + h * per_half, per_half)], idx_all)
            for b in range(LOOKAHEAD):
                gather(b, b)
            if h == 0:
                plsc.subcore_barrier()

            @pl.loop(0, per_half, step=NBUF)
            def _(jj):
                for b in range(NBUF):
                    j = jj + b
                    gwait(j, b)
                    scat(j, b)
                    jn = j + LOOKAHEAD
                    bn = (b + LOOKAHEAD) % NBUF

                    @pl.when(jn < per_half)
                    def _():
                        @pl.when(jn >= NBUF)
                        def _():
                            swait(jn - NBUF, bn)
                        gather(jn, bn)

            for b in range(NBUF):
                swait(per_half - NBUF + b, b)

        plsc.subcore_barrier()
        _striped_copy(s, lambda o, l: acc.at[pl.ds(o, l)],
                      lambda o, l: out_hbm.at[c, pl.ds(o, l)], n_nodes)

    return agg


def _make_hist(n_nodes, n_chunks):
    n_pad = n_nodes + NS
    per_core = n_chunks // NC
    per_sub = per_core // NS

    cp = pltpu.CompilerParams()
    if "needs_layout_passes" in pltpu.CompilerParams.__dataclass_fields__:
        cp = dataclasses.replace(cp, needs_layout_passes=False)

    @functools.partial(
        pl.kernel,
        out_type=jax.ShapeDtypeStruct((NC * NS, n_pad), jnp.float32),
        mesh=_sc_mesh(),
        compiler_params=cp,
        scratch_types=[
            pltpu.VMEM((2, CHUNK), jnp.int32),
            pltpu.VMEM((n_pad,), jnp.float32),
        ],
    )
    def hist(idx_hbm, zeros_hbm, out_hbm, idx_v, lhist):
        c = lax.axis_index("c")
        s = lax.axis_index("s")
        pltpu.sync_copy(zeros_hbm, lhist)

        base = c * per_core + s * per_sub
        ones = jnp.full((16,), 1.0, jnp.float32)

        @pl.loop(0, per_sub)
        def _(j):
            pltpu.sync_copy(idx_hbm.at[base + j], idx_v)
            for l in range(CHUNK // 16):
                colv = idx_v[1, pl.ds(l * 16, 16)]
                plsc.addupdate_scatter(lhist, [colv], ones)

        pltpu.sync_copy(lhist, out_hbm.at[c * NS + s])

    return hist


def _mm1_body(x_ref, w_ref, o_ref):
    o_ref[...] = jax.lax.dot_general(
        x_ref[...], w_ref[...], (((1,), (0,)), ((), ())),
        precision=lax.Precision.HIGHEST, preferred_element_type=jnp.float32)


def _scale_body(h_ref, hist_ref, g_ref, dis_ref):
    deg = jnp.sum(hist_ref[...], axis=1, keepdims=True) + 1.0
    dis = lax.rsqrt(deg)
    dis_ref[...] = dis
    g_ref[...] = h_ref[...] * dis


def _post1_body(p_ref, g_ref, dis_ref, b_ref, w_ref, o_ref):
    dis = dis_ref[...]
    t = dis * (p_ref[0] + p_ref[1] + g_ref[...]) + b_ref[...]
    t = jnp.maximum(t, 0.0)
    h2 = jax.lax.dot_general(
        t, w_ref[...], (((1,), (0,)), ((), ())),
        precision=lax.Precision.HIGHEST, preferred_element_type=jnp.float32)
    d_out = h2.shape[1]
    o_ref[:, :d_out] = h2 * dis
    o_ref[:, d_out:] = jnp.zeros_like(h2)


def _final_body(p_ref, g_ref, dis_ref, b_ref, o_ref):
    d_out = o_ref.shape[1]
    o = dis_ref[...] * (p_ref[0, :, :d_out] + p_ref[1, :, :d_out]
                        + g_ref[:, :d_out]) + b_ref[...]
    m = jnp.max(o, axis=1, keepdims=True)
    z = o - m
    lse = jnp.log(jnp.sum(jnp.exp(z), axis=1, keepdims=True))
    o_ref[...] = z - lse


def kernel(x, edge_index, W1, b1, W2, b2):
    n, d_in = x.shape
    d_hid = W1.shape[1]
    d_out = W2.shape[1]
    e = edge_index.shape[1]

    cpw = CHUNK * NC * NS * PAIRS_STEP
    e_pad = ((e + cpw - 1) // cpw) * cpw
    pad = e_pad - e
    n_chunks = e_pad // CHUNK
    ar = jnp.arange(pad, dtype=edge_index.dtype)
    row_p = jnp.concatenate([edge_index[0], ar % n])
    col_p = jnp.concatenate([edge_index[1], n + (ar % NS)])
    idx = jnp.stack([row_p.reshape(n_chunks, CHUNK),
                     col_p.reshape(n_chunks, CHUNK)], axis=1).astype(jnp.int32)

    z_rows, _ = _stripe(n + NS)
    zeros_hid = jnp.zeros((z_rows, d_hid), jnp.float32)
    zeros_flat = jnp.zeros((n + NS,), jnp.float32)

    hist32 = _make_hist(n, n_chunks)(idx, zeros_flat)
    hist_t = jnp.transpose(hist32)[:n]

    h1 = pl.pallas_call(
        _mm1_body,
        out_shape=jax.ShapeDtypeStruct((n, d_hid), jnp.float32),
    )(x, W1)

    g1, dis = pl.pallas_call(
        _scale_body,
        out_shape=(jax.ShapeDtypeStruct((n, d_hid), jnp.float32),
                   jax.ShapeDtypeStruct((n, 1), jnp.float32)),
    )(h1, hist_t)

    p1 = _make_agg(n, d_hid, n_chunks)(g1, idx, zeros_hid)

    g2 = pl.pallas_call(
        _post1_body,
        out_shape=jax.ShapeDtypeStruct((n, d_hid), jnp.float32),
    )(p1, g1, dis, b1.reshape(1, d_hid), W2)

    p2 = _make_agg(n, d_hid, n_chunks)(g2, idx, zeros_hid)

    out = pl.pallas_call(
        _final_body,
        out_shape=jax.ShapeDtypeStruct((n, d_out), jnp.float32),
    )(p2, g2, dis, b2.reshape(1, d_out))

    return out


# speedup vs baseline: 27.6553x; 1.1079x over previous
import dataclasses
import functools

import jax
import jax.numpy as jnp
from jax import lax
from jax.experimental import pallas as pl
from jax.experimental.pallas import tpu as pltpu
from jax.experimental.pallas import tpu_sc as plsc

NC = 2
NS = 16
CHUNK = 128
NBUF = 2
LOOKAHEAD = 1
NIDX = 2
PAIRS_STEP = NBUF * NIDX


def _sc_mesh():
    return plsc.VectorSubcoreMesh(core_axis_name="c", subcore_axis_name="s")


def _stripe(total):
    main = (total // NS) // 8 * 8
    tail = total - main * NS
    return main, tail


def _striped_copy(s, src_at, dst_at, total):
    main, tail = _stripe(total)
    pltpu.sync_copy(src_at(s * main, main), dst_at(s * main, main))
    if tail:
        @pl.when(s == NS - 1)
        def _():
            pltpu.sync_copy(src_at(NS * main, tail), dst_at(NS * main, tail))


def _make_agg(n_nodes, d, n_chunks):
    n_pad = n_nodes + NS
    per_core = n_chunks // NC
    per_sub = per_core // NS

    assert per_sub % (NBUF * NIDX) == 0
    per_half = per_sub // NIDX

    @functools.partial(
        pl.kernel,
        out_type=jax.ShapeDtypeStruct((NC, n_nodes, d), jnp.float32),
        mesh=_sc_mesh(),
        scratch_types=(
            [pltpu.VMEM((per_half, 2, CHUNK), jnp.int32)]
            + [pltpu.VMEM((CHUNK, d), jnp.float32)] * NBUF
            + [pltpu.VMEM_SHARED((n_pad, d), jnp.float32)]
            + [pltpu.SemaphoreType.DMA] * (2 * NBUF)
        ),
    )
    def agg(g_hbm, idx_hbm, zeros_hbm, out_hbm, idx_all, *scr):
        rows = scr[:NBUF]
        acc = scr[NBUF]
        gsems = scr[NBUF + 1:2 * NBUF + 1]
        ssems = scr[2 * NBUF + 1:]
        c = lax.axis_index("c")
        s = lax.axis_index("s")
        base = c * per_core + s * per_sub

        _striped_copy(s, lambda o, l: zeros_hbm.at[pl.ds(0, l)],
                      lambda o, l: acc.at[pl.ds(o, l)], n_pad)

        def gather(j, b):
            pltpu.async_copy(g_hbm.at[idx_all.at[j, 0]], rows[b], gsems[b])

        def gwait(j, b):
            pltpu.make_async_copy(
                g_hbm.at[idx_all.at[j, 0]], rows[b], gsems[b]).wait()

        def scat(j, b):
            pltpu.async_copy(rows[b], acc.at[idx_all.at[j, 1]], ssems[b],
                             add=True)

        def swait(j, b):
            pltpu.make_async_copy(
                rows[b], acc.at[idx_all.at[j, 1]], ssems[b]).wait()

        for h in range(NIDX):
            pltpu.sync_copy(
                idx_hbm.at[pl.ds(base + h * per_half, per_half)], idx_all)
            for b in range(LOOKAHEAD):
                gather(b, b)
            if h == 0:
                plsc.subcore_barrier()

            @pl.loop(0, per_half, step=NBUF)
            def _(jj):
                for b in range(NBUF):
                    j = jj + b
                    gwait(j, b)
                    scat(j, b)
                    jn = j + LOOKAHEAD
                    bn = (b + LOOKAHEAD) % NBUF

                    @pl.when(jn < per_half)
                    def _():
                        @pl.when(jn >= NBUF)
                        def _():
                            swait(jn - NBUF, bn)
                        gather(jn, bn)

            for b in range(NBUF):
                swait(per_half - NBUF + b, b)

        plsc.subcore_barrier()
        _striped_copy(s, lambda o, l: acc.at[pl.ds(o, l)],
                      lambda o, l: out_hbm.at[c, pl.ds(o, l)], n_nodes)

    return agg


def _make_hist(n_nodes, n_chunks):
    n_pad = n_nodes + NS
    per_core = n_chunks // NC
    per_sub = per_core // NS

    cp = pltpu.CompilerParams()
    if "needs_layout_passes" in pltpu.CompilerParams.__dataclass_fields__:
        cp = dataclasses.replace(cp, needs_layout_passes=False)
    per_half = per_sub // NIDX

    @functools.partial(
        pl.kernel,
        out_type=jax.ShapeDtypeStruct((NC * NS, n_pad), jnp.float32),
        mesh=_sc_mesh(),
        compiler_params=cp,
        scratch_types=[
            pltpu.VMEM((per_half, 2, CHUNK), jnp.int32),
            pltpu.VMEM((n_pad,), jnp.float32),
        ],
    )
    def hist(idx_hbm, zeros_hbm, out_hbm, idx_all, lhist):
        c = lax.axis_index("c")
        s = lax.axis_index("s")
        pltpu.sync_copy(zeros_hbm, lhist)

        base = c * per_core + s * per_sub
        ones = jnp.full((16,), 1.0, jnp.float32)

        for h in range(NIDX):
            pltpu.sync_copy(
                idx_hbm.at[pl.ds(base + h * per_half, per_half)], idx_all)

            @pl.loop(0, per_half)
            def _(j):
                for l in range(CHUNK // 16):
                    colv = idx_all[j, 1, pl.ds(l * 16, 16)]
                    plsc.addupdate_scatter(lhist, [colv], ones)

        pltpu.sync_copy(lhist, out_hbm.at[c * NS + s])

    return hist


def _mm1_body(x_ref, w_ref, o_ref):
    o_ref[...] = jax.lax.dot_general(
        x_ref[...], w_ref[...], (((1,), (0,)), ((), ())),
        precision=lax.Precision.HIGHEST, preferred_element_type=jnp.float32)


def _scale_body(h_ref, hist_ref, g_ref, dis_ref):
    deg = jnp.sum(hist_ref[...], axis=1, keepdims=True) + 1.0
    dis = lax.rsqrt(deg)
    dis_ref[...] = dis
    g_ref[...] = h_ref[...] * dis


def _post1_body(p_ref, g_ref, dis_ref, b_ref, w_ref, o_ref):
    dis = dis_ref[...]
    t = dis * (p_ref[0] + p_ref[1] + g_ref[...]) + b_ref[...]
    t = jnp.maximum(t, 0.0)
    h2 = jax.lax.dot_general(
        t, w_ref[...], (((1,), (0,)), ((), ())),
        precision=lax.Precision.HIGHEST, preferred_element_type=jnp.float32)
    d_out = h2.shape[1]
    o_ref[:, :d_out] = h2 * dis
    o_ref[:, d_out:] = jnp.zeros_like(h2)


def _final_body(p_ref, g_ref, dis_ref, b_ref, o_ref):
    d_out = o_ref.shape[1]
    o = dis_ref[...] * (p_ref[0, :, :d_out] + p_ref[1, :, :d_out]
                        + g_ref[:, :d_out]) + b_ref[...]
    m = jnp.max(o, axis=1, keepdims=True)
    z = o - m
    lse = jnp.log(jnp.sum(jnp.exp(z), axis=1, keepdims=True))
    o_ref[...] = z - lse


def kernel(x, edge_index, W1, b1, W2, b2):
    n, d_in = x.shape
    d_hid = W1.shape[1]
    d_out = W2.shape[1]
    e = edge_index.shape[1]

    cpw = CHUNK * NC * NS * PAIRS_STEP
    e_pad = ((e + cpw - 1) // cpw) * cpw
    pad = e_pad - e
    n_chunks = e_pad // CHUNK
    ar = jnp.arange(pad, dtype=edge_index.dtype)
    row_p = jnp.concatenate([edge_index[0], ar % n])
    col_p = jnp.concatenate([edge_index[1], n + (ar % NS)])
    idx = jnp.stack([row_p.reshape(n_chunks, CHUNK),
                     col_p.reshape(n_chunks, CHUNK)], axis=1).astype(jnp.int32)

    z_rows, _ = _stripe(n + NS)
    zeros_hid = jnp.zeros((z_rows, d_hid), jnp.float32)
    zeros_flat = jnp.zeros((n + NS,), jnp.float32)

    hist32 = _make_hist(n, n_chunks)(idx, zeros_flat)
    hist_t = jnp.transpose(hist32)[:n]

    h1 = pl.pallas_call(
        _mm1_body,
        out_shape=jax.ShapeDtypeStruct((n, d_hid), jnp.float32),
    )(x, W1)

    g1, dis = pl.pallas_call(
        _scale_body,
        out_shape=(jax.ShapeDtypeStruct((n, d_hid), jnp.float32),
                   jax.ShapeDtypeStruct((n, 1), jnp.float32)),
    )(h1, hist_t)

    p1 = _make_agg(n, d_hid, n_chunks)(g1, idx, zeros_hid)

    g2 = pl.pallas_call(
        _post1_body,
        out_shape=jax.ShapeDtypeStruct((n, d_hid), jnp.float32),
    )(p1, g1, dis, b1.reshape(1, d_hid), W2)

    p2 = _make_agg(n, d_hid, n_chunks)(g2, idx, zeros_hid)

    out = pl.pallas_call(
        _final_body,
        out_shape=jax.ShapeDtypeStruct((n, d_out), jnp.float32),
    )(p2, g2, dis, b2.reshape(1, d_out))

    return out


# speedup vs baseline: 27.8447x; 1.0068x over previous
import dataclasses
import functools

import jax
import jax.numpy as jnp
from jax import lax
from jax.experimental import pallas as pl
from jax.experimental.pallas import tpu as pltpu
from jax.experimental.pallas import tpu_sc as plsc

NC = 2
NS = 16
CHUNK = 64
NBUF = 4
LOOKAHEAD = 2
NIDX = 4
PAIRS_STEP = NBUF * NIDX


def _sc_mesh():
    return plsc.VectorSubcoreMesh(core_axis_name="c", subcore_axis_name="s")


def _stripe(total):
    main = (total // NS) // 8 * 8
    tail = total - main * NS
    return main, tail


def _striped_copy(s, src_at, dst_at, total):
    main, tail = _stripe(total)
    pltpu.sync_copy(src_at(s * main, main), dst_at(s * main, main))
    if tail:
        @pl.when(s == NS - 1)
        def _():
            pltpu.sync_copy(src_at(NS * main, tail), dst_at(NS * main, tail))


def _make_agg(n_nodes, d, n_chunks):
    n_pad = n_nodes + NS
    per_core = n_chunks // NC
    per_sub = per_core // NS

    assert per_sub % (NBUF * NIDX) == 0
    per_half = per_sub // NIDX

    @functools.partial(
        pl.kernel,
        out_type=jax.ShapeDtypeStruct((NC, n_nodes, d), jnp.float32),
        mesh=_sc_mesh(),
        scratch_types=(
            [pltpu.VMEM((per_half, 2, CHUNK), jnp.int32)]
            + [pltpu.VMEM((CHUNK, d), jnp.float32)] * NBUF
            + [pltpu.VMEM_SHARED((n_pad, d), jnp.float32)]
            + [pltpu.SemaphoreType.DMA] * (2 * NBUF)
        ),
    )
    def agg(g_hbm, idx_hbm, zeros_hbm, out_hbm, idx_all, *scr):
        rows = scr[:NBUF]
        acc = scr[NBUF]
        gsems = scr[NBUF + 1:2 * NBUF + 1]
        ssems = scr[2 * NBUF + 1:]
        c = lax.axis_index("c")
        s = lax.axis_index("s")
        base = c * per_core + s * per_sub

        _striped_copy(s, lambda o, l: zeros_hbm.at[pl.ds(0, l)],
                      lambda o, l: acc.at[pl.ds(o, l)], n_pad)

        def gather(j, b):
            pltpu.async_copy(g_hbm.at[idx_all.at[j, 0]], rows[b], gsems[b])

        def gwait(j, b):
            pltpu.make_async_copy(
                g_hbm.at[idx_all.at[j, 0]], rows[b], gsems[b]).wait()

        def scat(j, b):
            pltpu.async_copy(rows[b], acc.at[idx_all.at[j, 1]], ssems[b],
                             add=True)

        def swait(j, b):
            pltpu.make_async_copy(
                rows[b], acc.at[idx_all.at[j, 1]], ssems[b]).wait()

        for h in range(NIDX):
            pltpu.sync_copy(
                idx_hbm.at[pl.ds(base + h * per_half, per_half)], idx_all)
            for b in range(LOOKAHEAD):
                gather(b, b)
            if h == 0:
                plsc.subcore_barrier()

            @pl.loop(0, per_half, step=NBUF)
            def _(jj):
                for b in range(NBUF):
                    j = jj + b
                    gwait(j, b)
                    scat(j, b)
                    jn = j + LOOKAHEAD
                    bn = (b + LOOKAHEAD) % NBUF

                    @pl.when(jn < per_half)
                    def _():
                        @pl.when(jn >= NBUF)
                        def _():
                            swait(jn - NBUF, bn)
                        gather(jn, bn)

            for b in range(NBUF):
                swait(per_half - NBUF + b, b)

        plsc.subcore_barrier()
        _striped_copy(s, lambda o, l: acc.at[pl.ds(o, l)],
                      lambda o, l: out_hbm.at[c, pl.ds(o, l)], n_nodes)

    return agg


def _make_hist(n_nodes, n_chunks):
    n_pad = n_nodes + NS
    per_core = n_chunks // NC
    per_sub = per_core // NS

    cp = pltpu.CompilerParams()
    if "needs_layout_passes" in pltpu.CompilerParams.__dataclass_fields__:
        cp = dataclasses.replace(cp, needs_layout_passes=False)
    per_half = per_sub // NIDX

    @functools.partial(
        pl.kernel,
        out_type=jax.ShapeDtypeStruct((NC * NS, n_pad), jnp.float32),
        mesh=_sc_mesh(),
        compiler_params=cp,
        scratch_types=[
            pltpu.VMEM((per_half, 2, CHUNK), jnp.int32),
            pltpu.VMEM((n_pad,), jnp.float32),
        ],
    )
    def hist(idx_hbm, zeros_hbm, out_hbm, idx_all, lhist):
        c = lax.axis_index("c")
        s = lax.axis_index("s")
        pltpu.sync_copy(zeros_hbm, lhist)

        base = c * per_core + s * per_sub
        ones = jnp.full((16,), 1.0, jnp.float32)

        for h in range(NIDX):
            pltpu.sync_copy(
                idx_hbm.at[pl.ds(base + h * per_half, per_half)], idx_all)

            @pl.loop(0, per_half)
            def _(j):
                for l in range(CHUNK // 16):
                    colv = idx_all[j, 1, pl.ds(l * 16, 16)]
                    plsc.addupdate_scatter(lhist, [colv], ones)

        pltpu.sync_copy(lhist, out_hbm.at[c * NS + s])

    return hist


def _mm1_body(x_ref, w_ref, o_ref):
    o_ref[...] = jax.lax.dot_general(
        x_ref[...], w_ref[...], (((1,), (0,)), ((), ())),
        precision=lax.Precision.HIGHEST, preferred_element_type=jnp.float32)


def _scale_body(h_ref, hist_ref, g_ref, dis_ref):
    deg = jnp.sum(hist_ref[...], axis=1, keepdims=True) + 1.0
    dis = lax.rsqrt(deg)
    dis_ref[...] = dis
    g_ref[...] = h_ref[...] * dis


def _post1_body(p_ref, g_ref, dis_ref, b_ref, w_ref, o_ref):
    dis = dis_ref[...]
    t = dis * (p_ref[0] + p_ref[1] + g_ref[...]) + b_ref[...]
    t = jnp.maximum(t, 0.0)
    h2 = jax.lax.dot_general(
        t, w_ref[...], (((1,), (0,)), ((), ())),
        precision=lax.Precision.HIGHEST, preferred_element_type=jnp.float32)
    d_out = h2.shape[1]
    o_ref[:, :d_out] = h2 * dis
    o_ref[:, d_out:] = jnp.zeros_like(h2)


def _final_body(p_ref, g_ref, dis_ref, b_ref, o_ref):
    d_out = o_ref.shape[1]
    o = dis_ref[...] * (p_ref[0, :, :d_out] + p_ref[1, :, :d_out]
                        + g_ref[:, :d_out]) + b_ref[...]
    m = jnp.max(o, axis=1, keepdims=True)
    z = o - m
    lse = jnp.log(jnp.sum(jnp.exp(z), axis=1, keepdims=True))
    o_ref[...] = z - lse


def kernel(x, edge_index, W1, b1, W2, b2):
    n, d_in = x.shape
    d_hid = W1.shape[1]
    d_out = W2.shape[1]
    e = edge_index.shape[1]

    cpw = CHUNK * NC * NS * PAIRS_STEP
    e_pad = ((e + cpw - 1) // cpw) * cpw
    pad = e_pad - e
    n_chunks = e_pad // CHUNK
    ar = jnp.arange(pad, dtype=edge_index.dtype)
    row_p = jnp.concatenate([edge_index[0], ar % n])
    col_p = jnp.concatenate([edge_index[1], n + (ar % NS)])
    idx = jnp.stack([row_p.reshape(n_chunks, CHUNK),
                     col_p.reshape(n_chunks, CHUNK)], axis=1).astype(jnp.int32)

    z_rows, _ = _stripe(n + NS)
    zeros_hid = jnp.zeros((z_rows, d_hid), jnp.float32)
    zeros_flat = jnp.zeros((n + NS,), jnp.float32)

    hist32 = _make_hist(n, n_chunks)(idx, zeros_flat)
    hist_t = jnp.transpose(hist32)[:n]

    h1 = pl.pallas_call(
        _mm1_body,
        out_shape=jax.ShapeDtypeStruct((n, d_hid), jnp.float32),
    )(x, W1)

    g1, dis = pl.pallas_call(
        _scale_body,
        out_shape=(jax.ShapeDtypeStruct((n, d_hid), jnp.float32),
                   jax.ShapeDtypeStruct((n, 1), jnp.float32)),
    )(h1, hist_t)

    p1 = _make_agg(n, d_hid, n_chunks)(g1, idx, zeros_hid)

    g2 = pl.pallas_call(
        _post1_body,
        out_shape=jax.ShapeDtypeStruct((n, d_hid), jnp.float32),
    )(p1, g1, dis, b1.reshape(1, d_hid), W2)

    p2 = _make_agg(n, d_hid, n_chunks)(g2, idx, zeros_hid)

    out = pl.pallas_call(
        _final_body,
        out_shape=jax.ShapeDtypeStruct((n, d_out), jnp.float32),
    )(p2, g2, dis, b2.reshape(1, d_out))

    return out
